# speedup vs baseline: 2.1753x; 2.1753x over previous
import functools

import jax
import jax.numpy as jnp
from jax import lax
from jax.experimental import pallas as pl
from jax.experimental.pallas import tpu as pltpu
from jax.experimental.pallas import tpu_sc as plsc

D_MODEL = 1024
NUM_CORES = 2
NUM_SUBCORES = 16
NUM_WORKERS = NUM_CORES * NUM_SUBCORES
TOTAL = 4 * 8192
PER_WORKER = TOTAL // NUM_WORKERS
CHUNK = 64
NUM_CHUNKS = PER_WORKER // CHUNK

_mesh = plsc.VectorSubcoreMesh(core_axis_name="c", subcore_axis_name="s")


@functools.partial(
    pl.kernel,
    mesh=_mesh,
    out_type=jax.ShapeDtypeStruct((TOTAL, D_MODEL), jnp.float32),
    scratch_types=[
        pltpu.VMEM((NUM_CHUNKS, CHUNK), jnp.int32),
        pltpu.VMEM((CHUNK, D_MODEL), jnp.float32),
        pltpu.SemaphoreType.DMA,
    ],
)
def _gather_kernel(pe_hbm, idx_hbm, out_hbm, idx_v, rows_v, sem):
    wid = lax.axis_index("s") * NUM_CORES + lax.axis_index("c")
    base = wid * PER_WORKER
    pltpu.sync_copy(idx_hbm.at[wid], idx_v)

    def body(c, carry):
        pltpu.async_copy(pe_hbm.at[idx_v.at[c]], rows_v, sem).wait()
        pltpu.sync_copy(rows_v, out_hbm.at[pl.ds(base + c * CHUNK, CHUNK)])
        return carry

    lax.fori_loop(0, NUM_CHUNKS, body, 0)


def kernel(position_ids, pe):
    idx = position_ids.reshape(NUM_WORKERS, NUM_CHUNKS, CHUNK).astype(jnp.int32)
    out = _gather_kernel(pe, idx)
    return out.reshape(position_ids.shape + (D_MODEL,))


# speedup vs baseline: 2.2882x; 1.0519x over previous
import functools

import jax
import jax.numpy as jnp
from jax import lax
from jax.experimental import pallas as pl
from jax.experimental.pallas import tpu as pltpu
from jax.experimental.pallas import tpu_sc as plsc

D_MODEL = 1024
NUM_CORES = 2
NUM_SUBCORES = 16
NUM_WORKERS = NUM_CORES * NUM_SUBCORES
TOTAL = 4 * 8192
PER_WORKER = TOTAL // NUM_WORKERS
CHUNK = 32
NUM_CHUNKS = PER_WORKER // CHUNK

_mesh = plsc.VectorSubcoreMesh(core_axis_name="c", subcore_axis_name="s")


@functools.partial(
    pl.kernel,
    mesh=_mesh,
    out_type=jax.ShapeDtypeStruct((TOTAL, D_MODEL), jnp.float32),
    scratch_types=[
        pltpu.VMEM((NUM_CHUNKS, CHUNK), jnp.int32),
        pltpu.VMEM((CHUNK, D_MODEL), jnp.float32),
        pltpu.VMEM((CHUNK, D_MODEL), jnp.float32),
        pltpu.SemaphoreType.DMA,
        pltpu.SemaphoreType.DMA,
        pltpu.SemaphoreType.DMA,
        pltpu.SemaphoreType.DMA,
    ],
)
def _gather_kernel(pe_hbm, idx_hbm, out_hbm, idx_v, buf0, buf1, g0, g1, s0, s1):
    wid = lax.axis_index("s") * NUM_CORES + lax.axis_index("c")
    base = wid * PER_WORKER
    pltpu.sync_copy(idx_hbm.at[wid], idx_v)

    def start_gather(c, buf, sem):
        pltpu.async_copy(pe_hbm.at[idx_v.at[c]], buf, sem)

    def wait_gather(c, buf, sem):
        pltpu.make_async_copy(pe_hbm.at[idx_v.at[c]], buf, sem).wait()

    def start_scatter(c, buf, sem):
        pltpu.async_copy(buf, out_hbm.at[pl.ds(base + c * CHUNK, CHUNK)], sem)

    def wait_scatter(c, buf, sem):
        pltpu.make_async_copy(
            buf, out_hbm.at[pl.ds(base + c * CHUNK, CHUNK)], sem
        ).wait()

    start_gather(0, buf0, g0)

    def body(i, carry):
        c0 = 2 * i
        wait_gather(c0, buf0, g0)
        start_gather(c0 + 1, buf1, g1)
        start_scatter(c0, buf0, s0)
        wait_gather(c0 + 1, buf1, g1)
        wait_scatter(c0, buf0, s0)
        start_gather(c0 + 2, buf0, g0)
        start_scatter(c0 + 1, buf1, s1)
        wait_scatter(c0 + 1, buf1, s1)
        return carry

    lax.fori_loop(0, NUM_CHUNKS // 2 - 1, body, 0)

    last = NUM_CHUNKS - 2
    wait_gather(last, buf0, g0)
    start_gather(last + 1, buf1, g1)
    start_scatter(last, buf0, s0)
    wait_gather(last + 1, buf1, g1)
    start_scatter(last + 1, buf1, s1)
    wait_scatter(last, buf0, s0)
    wait_scatter(last + 1, buf1, s1)


def kernel(position_ids, pe):
    idx = position_ids.reshape(NUM_WORKERS, NUM_CHUNKS, CHUNK).astype(jnp.int32)
    out = _gather_kernel(pe, idx)
    return out.reshape(position_ids.shape + (D_MODEL,))
